# Initial kernel scaffold; baseline (speedup 1.0000x reference)
import jax
import jax.numpy as jnp
from jax.experimental import pallas as pl


def kernel(input, log_delay, log_weight):
    raise NotImplementedError("write your pallas kernel here")


# speedup vs baseline: 278.8253x; 278.8253x over previous
import jax
import jax.numpy as jnp
from jax.experimental import pallas as pl
from jax.experimental.pallas import tpu as pltpu


def _tc_body(xt_ref, rd_ref, lw_ref, out_ref):
    T = xt_ref.shape[3]
    C = xt_ref.shape[2]
    D = rd_ref.shape[3]

    w = jnp.exp(lw_ref[0, 0])

    def one_component(j):
        x = xt_ref[j, 0, :, :]
        m = jnp.max(x, axis=1, keepdims=True)
        tio = jax.lax.broadcasted_iota(jnp.int32, (C, T), 1)
        argm = jnp.min(jnp.where(x == m, tio, T), axis=1)
        cap = (T - 1) - argm
        rd = jnp.minimum(rd_ref[0, j, :, :], cap[:, None]) & (T - 1)
        xt = x.T
        o = jnp.broadcast_to(xt[:, :, None], (T, C, D))
        bit = 1
        while bit < T:
            rolled = jnp.concatenate([o[T - bit:], o[:T - bit]], axis=0)
            mask = ((rd & bit) != 0)[None, :, :]
            o = jnp.where(mask, rolled, o)
            bit *= 2
        return o

    out_ref[:, 0, :, :] = (one_component(0) + one_component(1)) * w


def _stochastic_round_delays(log_delay, N, C):
    D = log_delay.shape[0]
    delay = jnp.concatenate([jnp.exp(log_delay), jnp.exp(log_delay[::-1])],
                            axis=1)
    db = jnp.broadcast_to(delay[None, None, :, :], (N, C, D, 2))
    fl = jnp.floor(db)
    p = db - fl
    bern = jax.random.bernoulli(jax.random.key(42), p)
    return jnp.where(bern, fl + 1.0, fl).astype(jnp.int32)


def kernel(input, log_delay, log_weight):
    T, N, C, _ = input.shape
    D = log_delay.shape[0]

    rd_pre = _stochastic_round_delays(log_delay, N, C)
    rd_t = jnp.transpose(rd_pre, (0, 3, 1, 2))
    xt = jnp.transpose(input, (3, 1, 2, 0))
    lw = jnp.reshape(log_weight, (1, 1)).astype(jnp.float32)

    out = pl.pallas_call(
        _tc_body,
        grid=(N,),
        in_specs=[
            pl.BlockSpec((2, 1, C, T), lambda n: (0, n, 0, 0)),
            pl.BlockSpec((1, 2, C, D), lambda n: (n, 0, 0, 0)),
            pl.BlockSpec(memory_space=pltpu.SMEM),
        ],
        out_specs=pl.BlockSpec((T, 1, C, D), lambda n: (0, n, 0, 0)),
        out_shape=jax.ShapeDtypeStruct((T, N, C, D), jnp.float32),
    )(xt, rd_t, lw)
    return out
